# baseline (speedup 1.0000x reference)
import jax
import jax.numpy as jnp
from jax.experimental import pallas as pl

_D = 1024
_SCALE = float(_D) ** 0.5
_BS = 256


def _ppl_kernel(seq_ref, pe_ref, out_ref):
    out_ref[...] = seq_ref[...] * _SCALE + pe_ref[...]


@jax.jit
def _run(sequence, pe):
    batch, seq_len, d = sequence.shape
    grid = (seq_len // _BS,)
    return pl.pallas_call(
        _ppl_kernel,
        grid=grid,
        in_specs=[
            pl.BlockSpec((batch, _BS, d), lambda i: (0, i, 0)),
            pl.BlockSpec((1, _BS, d), lambda i: (0, i, 0)),
        ],
        out_specs=pl.BlockSpec((batch, _BS, d), lambda i: (0, i, 0)),
        out_shape=jax.ShapeDtypeStruct((batch, seq_len, d), sequence.dtype),
    )(sequence, pe)


def kernel(sequence, pe, training, mask):
    del training, mask
    seq_len = sequence.shape[1]
    return _run(sequence, pe[:, :seq_len, :])
